# Initial kernel scaffold; baseline (speedup 1.0000x reference)
import functools

import jax
import jax.numpy as jnp
from jax import lax
from jax.experimental import pallas as pl
from jax.experimental.pallas import tpu as pltpu
from jax.experimental.pallas import tpu_sc as plsc

MAXLEN = 200
EMBED = 32
LANES = 16
NC = 2
NS = 16
NW = NC * NS
C = 400
NBUF = 4
DG = 2


@functools.lru_cache(maxsize=None)
def _make_emb(B):
    BPW = B // NW
    NCH = BPW // C
    assert B % NW == 0 and BPW % C == 0 and NCH % NBUF == 0 and C % MAXLEN == 0

    mesh = plsc.VectorSubcoreMesh(core_axis_name="c", subcore_axis_name="s")

    @functools.partial(
        pl.kernel,
        mesh=mesh,
        out_type=jax.ShapeDtypeStruct((B, EMBED), jnp.float32),
        scratch_types=(
            [pltpu.VMEM((BPW,), jnp.int32),
             pltpu.VMEM((MAXLEN, EMBED), jnp.float32)]
            + [pltpu.VMEM((C, EMBED), jnp.float32) for _ in range(NBUF)]
            + [pltpu.SemaphoreType.DMA for _ in range(2 * NBUF + 1)]
        ),
    )
    def emb(x_hbm, tok_hbm, pos_hbm, out_hbm, idx_v, pat_v, *rest):
        rows = rest[:NBUF]
        gsems = rest[NBUF:2 * NBUF]
        ssems = rest[2 * NBUF:3 * NBUF]
        lsem = rest[3 * NBUF]

        wid = lax.axis_index("s") * NC + lax.axis_index("c")
        base = wid * BPW

        pltpu.async_copy(x_hbm.at[pl.ds(base, BPW)], idx_v, lsem).wait()
        pltpu.async_copy(pos_hbm, pat_v, lsem).wait()

        def start_gather(g, b):
            pltpu.make_async_copy(
                tok_hbm.at[idx_v.at[pl.ds(g * C, C)]], rows[b], gsems[b]
            ).start()

        def wait_gather(b):
            pltpu.make_async_copy(
                tok_hbm.at[idx_v.at[pl.ds(0, C)]], rows[b], gsems[b]
            ).wait()

        def start_scatter(g, b):
            pltpu.make_async_copy(
                rows[b], out_hbm.at[pl.ds(base + g * C, C)], ssems[b]
            ).start()

        def wait_scatter(b):
            pltpu.make_async_copy(
                rows[b], out_hbm.at[pl.ds(base, C)], ssems[b]
            ).wait()

        def add_pattern(rows_ref):
            for j in range(C // MAXLEN):
                def body(r, carry, j=j):
                    rr = j * MAXLEN + r
                    plsc.addupdate(rows_ref.at[rr, pl.ds(0, LANES)],
                                   pat_v[r, pl.ds(0, LANES)])
                    plsc.addupdate(rows_ref.at[rr, pl.ds(LANES, LANES)],
                                   pat_v[r, pl.ds(LANES, LANES)])
                    return carry
                lax.fori_loop(0, MAXLEN, body, 0)

        for b in range(DG):
            start_gather(b, b)

        def outer(i, carry):
            g0 = i * NBUF
            for b in range(NBUF):
                g = g0 + b
                nxt = g + DG
                bn = (b + DG) % NBUF

                @pl.when(nxt < NCH)
                def _(g=g, nxt=nxt, bn=bn):
                    @pl.when(nxt >= NBUF)
                    def _():
                        wait_scatter(bn)
                    start_gather(nxt, bn)

                wait_gather(b)
                add_pattern(rows[b])
                start_scatter(g, b)
            return carry

        lax.fori_loop(0, NCH // NBUF, outer, 0)

        for b in range(NBUF):
            wait_scatter(b)

    return emb


def kernel(x, token_table, pos_table):
    batch, maxlen = x.shape
    B = batch * maxlen
    xf = x.reshape(B).astype(jnp.int32)
    out = _make_emb(B)(xf, token_table, pos_table)
    return out.reshape(batch, maxlen, EMBED)


# speedup vs baseline: 1.4676x; 1.4676x over previous
import functools

import jax
import jax.numpy as jnp
from jax import lax
from jax.experimental import pallas as pl
from jax.experimental.pallas import tpu as pltpu
from jax.experimental.pallas import tpu_sc as plsc

MAXLEN = 200
EMBED = 32
LANES = 16
NC = 2
NS = 16
NW = NC * NS
C = 400
NBUF = 4
DG = 2


@functools.lru_cache(maxsize=None)
def _make_emb(B):
    BPW = B // NW
    NCH = BPW // C
    assert B % NW == 0 and BPW % C == 0 and NCH % NBUF == 0 and C % MAXLEN == 0

    mesh = plsc.VectorSubcoreMesh(core_axis_name="c", subcore_axis_name="s")

    @functools.partial(
        pl.kernel,
        mesh=mesh,
        compiler_params=pltpu.CompilerParams(use_tc_tiling_on_sc=False),
        out_type=jax.ShapeDtypeStruct((B, EMBED), jnp.float32),
        scratch_types=(
            [pltpu.VMEM((BPW,), jnp.int32),
             pltpu.VMEM((MAXLEN, EMBED), jnp.float32)]
            + [pltpu.VMEM((C, EMBED), jnp.float32) for _ in range(NBUF)]
            + [pltpu.SemaphoreType.DMA for _ in range(2 * NBUF + 1)]
        ),
    )
    def emb(x_hbm, tok_hbm, pos_hbm, out_hbm, idx_v, pat_v, *rest):
        rows = rest[:NBUF]
        gsems = rest[NBUF:2 * NBUF]
        ssems = rest[2 * NBUF:3 * NBUF]
        lsem = rest[3 * NBUF]

        wid = lax.axis_index("s") * NC + lax.axis_index("c")
        base = wid * BPW

        pltpu.async_copy(x_hbm.at[pl.ds(base, BPW)], idx_v, lsem).wait()
        pltpu.async_copy(pos_hbm, pat_v, lsem).wait()

        def start_gather(g, b):
            pltpu.make_async_copy(
                tok_hbm.at[idx_v.at[pl.ds(g * C, C)]], rows[b], gsems[b]
            ).start()

        def wait_gather(b):
            pltpu.make_async_copy(
                tok_hbm.at[idx_v.at[pl.ds(0, C)]], rows[b], gsems[b]
            ).wait()

        def start_scatter(g, b):
            pltpu.make_async_copy(
                rows[b], out_hbm.at[pl.ds(base + g * C, C)], ssems[b]
            ).start()

        def wait_scatter(b):
            pltpu.make_async_copy(
                rows[b], out_hbm.at[pl.ds(base, C)], ssems[b]
            ).wait()

        def add_pattern(rows_ref):
            for j in range(C // MAXLEN):
                def body(r, carry, j=j):
                    rr = j * MAXLEN + r
                    plsc.addupdate(rows_ref.at[rr, pl.ds(0, LANES)],
                                   pat_v[r, pl.ds(0, LANES)])
                    plsc.addupdate(rows_ref.at[rr, pl.ds(LANES, LANES)],
                                   pat_v[r, pl.ds(LANES, LANES)])
                    return carry
                lax.fori_loop(0, MAXLEN, body, 0)

        for b in range(DG):
            start_gather(b, b)

        def outer(i, carry):
            g0 = i * NBUF
            for b in range(NBUF):
                g = g0 + b
                nxt = g + DG
                bn = (b + DG) % NBUF

                @pl.when(nxt < NCH)
                def _(g=g, nxt=nxt, bn=bn):
                    @pl.when(nxt >= NBUF)
                    def _():
                        wait_scatter(bn)
                    start_gather(nxt, bn)

                wait_gather(b)
                add_pattern(rows[b])
                start_scatter(g, b)
            return carry

        lax.fori_loop(0, NCH // NBUF, outer, 0)

        for b in range(NBUF):
            wait_scatter(b)

    return emb


def kernel(x, token_table, pos_table):
    batch, maxlen = x.shape
    B = batch * maxlen
    xf = x.reshape(B).astype(jnp.int32)
    out = _make_emb(B)(xf, token_table, pos_table)
    return out.reshape(batch, maxlen, EMBED)
